# baseline (speedup 1.0000x reference)
import functools

import jax
import jax.numpy as jnp
from jax import lax
from jax.experimental import pallas as pl
from jax.experimental.pallas import tpu as pltpu
from jax.experimental.pallas import tpu_sc as plsc

ED = 64
WD = 128
M = 1000
ROWS = M + 1
BATCH = 1024
NB = 8
NIDX = BATCH * NB * NB * NB

NC = 2
NS = 16
NW = NC * NS
NI = NIDX // NW
ISZ = 128
NSTREAM = 2
CHUNK = ISZ * NSTREAM
NBUF = 4
NCHUNK = NI // CHUNK


def _body(table_hbm, gi_hbm, out_hbm, table_sh, idx_all, rows, *sems):
    c = lax.axis_index("c")
    s = lax.axis_index("s")
    wid = s * NC + c
    base = wid * NI
    base_row = wid * (NI // ISZ)

    @pl.when(s == 0)
    def _stage():
        pltpu.sync_copy(table_hbm.at[:, pl.ds(0, ED)], table_sh)

    pltpu.sync_copy(gi_hbm.at[pl.ds(base_row, NI // ISZ)], idx_all)
    plsc.subcore_barrier()

    def fire(t, b):
        for j in range(NSTREAM):
            pltpu.async_copy(
                table_sh.at[idx_all.at[t * NSTREAM + j]],
                rows.at[b, pl.ds(j * ISZ, ISZ)],
                sems[b],
            )

    def drain(t, b):
        for j in range(NSTREAM):
            pltpu.make_async_copy(
                table_sh.at[idx_all.at[t * NSTREAM + j]],
                rows.at[b, pl.ds(j * ISZ, ISZ)],
                sems[b],
            ).wait()

    for b in range(NBUF):
        fire(b, b)

    def outer(tt, carry):
        for b in range(NBUF):
            t = tt * NBUF + b
            drain(t, b)
            pltpu.sync_copy(
                rows.at[b],
                out_hbm.at[pl.ds(base + t * CHUNK, CHUNK), pl.ds(0, ED)],
            )
            nt = t + NBUF

            @pl.when(nt < NCHUNK)
            def _():
                fire(nt, b)

        return carry

    lax.fori_loop(0, NCHUNK // NBUF, outer, 0)


@jax.jit
def _gather(table, gi_flat):
    mesh = plsc.VectorSubcoreMesh(core_axis_name="c", subcore_axis_name="s")
    f = functools.partial(
        pl.kernel,
        mesh=mesh,
        out_type=jax.ShapeDtypeStruct((NIDX, WD), jnp.float32),
        scratch_types=[
            pltpu.VMEM_SHARED((ROWS, ED), jnp.float32),
            pltpu.VMEM((NI // ISZ, ISZ), jnp.int32),
            pltpu.VMEM((NBUF, CHUNK, ED), jnp.float32),
        ] + [pltpu.SemaphoreType.DMA] * NBUF,
        compiler_params=pltpu.CompilerParams(use_tc_tiling_on_sc=False),
    )(_body)
    return f(table, gi_flat)


TCR = 8192


def _tc_slice_body(i_ref, o_ref):
    o_ref[...] = i_ref[:, :ED]


def _tc_slice(out128):
    return pl.pallas_call(
        _tc_slice_body,
        grid=(NIDX // TCR,),
        in_specs=[pl.BlockSpec((TCR, WD), lambda i: (i, 0))],
        out_specs=pl.BlockSpec((TCR, ED), lambda i: (i, 0)),
        out_shape=jax.ShapeDtypeStruct((NIDX, ED), jnp.float32),
    )(out128)


def kernel(x, gi, ee):
    table = jnp.concatenate([ee, x], axis=0)
    table = jnp.pad(table, ((0, 0), (0, WD - ED)))
    gi_flat = gi.reshape(NIDX // ISZ, ISZ).astype(jnp.int32)
    out = _gather(table, gi_flat)
    return _tc_slice(out).reshape(BATCH, NB, NB, NB, ED)


# speedup vs baseline: 1.8668x; 1.8668x over previous
import functools

import jax
import jax.numpy as jnp
from jax import lax
from jax.experimental import pallas as pl
from jax.experimental.pallas import tpu as pltpu
from jax.experimental.pallas import tpu_sc as plsc

ED = 64
WD = 128
M = 1000
ROWS = M + 1
BATCH = 1024
NB = 8
NIDX = BATCH * NB * NB * NB

NC = 2
NS = 16
NW = NC * NS
NI = NIDX // NW
ISZ = 128
NSTREAM = 4
CHUNK = ISZ * NSTREAM
NBUF = 2
NCHUNK = NI // CHUNK


def _body(table_hbm, gi_hbm, out_hbm, table_sh, idx_all, rows, *sems):
    c = lax.axis_index("c")
    s = lax.axis_index("s")
    wid = s * NC + c
    base = wid * NI
    base_row = wid * (NI // ISZ)

    @pl.when(s == 0)
    def _stage():
        pltpu.sync_copy(table_hbm.at[:, pl.ds(0, ED)], table_sh)

    pltpu.sync_copy(gi_hbm.at[pl.ds(base_row, NI // ISZ)], idx_all)
    plsc.subcore_barrier()

    def fire(t, b):
        for j in range(NSTREAM):
            pltpu.async_copy(
                table_sh.at[idx_all.at[t * NSTREAM + j]],
                rows.at[b, pl.ds(j * ISZ, ISZ)],
                sems[b],
            )

    def drain(t, b):
        for j in range(NSTREAM):
            pltpu.make_async_copy(
                table_sh.at[idx_all.at[t * NSTREAM + j]],
                rows.at[b, pl.ds(j * ISZ, ISZ)],
                sems[b],
            ).wait()

    for b in range(NBUF):
        fire(b, b)

    def outer(tt, carry):
        for b in range(NBUF):
            t = tt * NBUF + b
            drain(t, b)
            pltpu.sync_copy(
                rows.at[b],
                out_hbm.at[pl.ds(base + t * CHUNK, CHUNK), pl.ds(0, ED)],
            )
            nt = t + NBUF

            @pl.when(nt < NCHUNK)
            def _():
                fire(nt, b)

        return carry

    lax.fori_loop(0, NCHUNK // NBUF, outer, 0)


@jax.jit
def _gather(table, gi_flat):
    mesh = plsc.VectorSubcoreMesh(core_axis_name="c", subcore_axis_name="s")
    f = functools.partial(
        pl.kernel,
        mesh=mesh,
        out_type=jax.ShapeDtypeStruct((NIDX, WD), jnp.float32),
        scratch_types=[
            pltpu.VMEM_SHARED((ROWS, ED), jnp.float32),
            pltpu.VMEM((NI // ISZ, ISZ), jnp.int32),
            pltpu.VMEM((NBUF, CHUNK, ED), jnp.float32),
        ] + [pltpu.SemaphoreType.DMA] * NBUF,
        compiler_params=pltpu.CompilerParams(use_tc_tiling_on_sc=False),
    )(_body)
    return f(table, gi_flat)


def kernel(x, gi, ee):
    table = jnp.concatenate([ee, x], axis=0)
    table = jnp.pad(table, ((0, 0), (0, WD - ED)))
    gi_flat = gi.reshape(NIDX // ISZ, ISZ).astype(jnp.int32)
    out = _gather(table, gi_flat)
    return out[:, :ED].reshape(BATCH, NB, NB, NB, ED)


# speedup vs baseline: 1.8926x; 1.0138x over previous
import functools

import jax
import jax.numpy as jnp
from jax import lax
from jax.experimental import pallas as pl
from jax.experimental.pallas import tpu as pltpu
from jax.experimental.pallas import tpu_sc as plsc

ED = 64
WD = 128
M = 1000
ROWS = M + 1
BATCH = 1024
NB = 8
NIDX = BATCH * NB * NB * NB

NC = 2
NS = 16
NW = NC * NS
NI = NIDX // NW
ISZ = 128
NSTREAM = 1
CHUNK = ISZ * NSTREAM
NBUF = 8
NCHUNK = NI // CHUNK


def _body(table_hbm, gi_hbm, out_hbm, table_sh, idx_all, rows, *sems):
    c = lax.axis_index("c")
    s = lax.axis_index("s")
    wid = s * NC + c
    base = wid * NI
    base_row = wid * (NI // ISZ)

    @pl.when(s == 0)
    def _stage():
        pltpu.sync_copy(table_hbm.at[:, pl.ds(0, ED)], table_sh)

    pltpu.sync_copy(gi_hbm.at[pl.ds(base_row, NI // ISZ)], idx_all)
    plsc.subcore_barrier()

    def fire(t, b):
        for j in range(NSTREAM):
            pltpu.async_copy(
                table_sh.at[idx_all.at[t * NSTREAM + j]],
                rows.at[b, pl.ds(j * ISZ, ISZ)],
                sems[b],
            )

    def drain(t, b):
        for j in range(NSTREAM):
            pltpu.make_async_copy(
                table_sh.at[idx_all.at[t * NSTREAM + j]],
                rows.at[b, pl.ds(j * ISZ, ISZ)],
                sems[b],
            ).wait()

    for b in range(NBUF):
        fire(b, b)

    def outer(tt, carry):
        for b in range(NBUF):
            t = tt * NBUF + b
            drain(t, b)
            pltpu.sync_copy(
                rows.at[b],
                out_hbm.at[pl.ds(base + t * CHUNK, CHUNK), pl.ds(0, ED)],
            )
            nt = t + NBUF

            @pl.when(nt < NCHUNK)
            def _():
                fire(nt, b)

        return carry

    lax.fori_loop(0, NCHUNK // NBUF, outer, 0)


@jax.jit
def _gather(table, gi_flat):
    mesh = plsc.VectorSubcoreMesh(core_axis_name="c", subcore_axis_name="s")
    f = functools.partial(
        pl.kernel,
        mesh=mesh,
        out_type=jax.ShapeDtypeStruct((NIDX, WD), jnp.float32),
        scratch_types=[
            pltpu.VMEM_SHARED((ROWS, ED), jnp.float32),
            pltpu.VMEM((NI // ISZ, ISZ), jnp.int32),
            pltpu.VMEM((NBUF, CHUNK, ED), jnp.float32),
        ] + [pltpu.SemaphoreType.DMA] * NBUF,
        compiler_params=pltpu.CompilerParams(use_tc_tiling_on_sc=False),
    )(_body)
    return f(table, gi_flat)


def kernel(x, gi, ee):
    table = jnp.concatenate([ee, x], axis=0)
    table = jnp.pad(table, ((0, 0), (0, WD - ED)))
    gi_flat = gi.reshape(NIDX // ISZ, ISZ).astype(jnp.int32)
    out = _gather(table, gi_flat)
    return out[:, :ED].reshape(BATCH, NB, NB, NB, ED)
